# baseline (speedup 1.0000x reference)
import functools

import jax
import jax.numpy as jnp
from jax import lax
from jax.experimental import pallas as pl
from jax.experimental.pallas import tpu as pltpu
from jax.experimental.pallas import tpu_sc as plsc

N = 10000
F_IN = 128
HID = 32

NC, NS = 2, 16
NW = NC * NS
B = 128
K = 80
EPW = K * B
E_PAD = NW * EPW
N_PAD = 10240
RPS = N_PAD // NS

_mesh = plsc.VectorSubcoreMesh(
    core_axis_name="c", subcore_axis_name="s", num_cores=NC, num_subcores=NS)


@functools.partial(
    pl.kernel,
    out_type=jax.ShapeDtypeStruct((NC, N_PAD), jnp.float32),
    mesh=_mesh,
    scratch_types=[
        pltpu.VMEM((K, B), jnp.int32),
        pltpu.VMEM((B,), jnp.float32),
        pltpu.VMEM((RPS,), jnp.float32),
        pltpu.VMEM_SHARED((N_PAD,), jnp.float32),
        pltpu.SemaphoreType.DMA,
    ],
    compiler_params=pltpu.CompilerParams(use_tc_tiling_on_sc=False),
)
def _sc_degree(dst_hbm, ones_hbm, zeros_hbm, out_hbm, dst_v, ones_v, stage_v,
               acc_sh, sem):
    c = lax.axis_index("c")
    s = lax.axis_index("s")
    wid = s * NC + c

    pltpu.sync_copy(dst_hbm.at[wid], dst_v)
    pltpu.sync_copy(ones_hbm, ones_v)
    pltpu.sync_copy(zeros_hbm, stage_v)
    pltpu.sync_copy(stage_v, acc_sh.at[pl.ds(s * RPS, RPS)])
    plsc.subcore_barrier()

    def fire(j, carry):
        pltpu.async_copy(ones_v, acc_sh.at[dst_v.at[j]], sem, add=True)
        return carry

    def drain(j, carry):
        pltpu.make_async_copy(ones_v, acc_sh.at[dst_v.at[j]], sem).wait()
        return carry

    lax.fori_loop(0, K, fire, 0)
    lax.fori_loop(0, K, drain, 0)
    plsc.subcore_barrier()

    pltpu.sync_copy(acc_sh.at[pl.ds(s * RPS, RPS)], stage_v)
    pltpu.sync_copy(stage_v, out_hbm.at[c, pl.ds(s * RPS, RPS)])


@functools.partial(
    pl.kernel,
    out_type=jax.ShapeDtypeStruct((NC, N_PAD, HID), jnp.float32),
    mesh=_mesh,
    scratch_types=[
        pltpu.VMEM((K, B), jnp.int32),
        pltpu.VMEM((K, B), jnp.int32),
        pltpu.VMEM((B, HID), jnp.float32),
        pltpu.VMEM((B, HID), jnp.float32),
        pltpu.VMEM((RPS, HID), jnp.float32),
        pltpu.VMEM_SHARED((N_PAD, HID), jnp.float32),
        pltpu.SemaphoreType.DMA,
        pltpu.SemaphoreType.DMA,
    ],
    compiler_params=pltpu.CompilerParams(use_tc_tiling_on_sc=False),
)
def _sc_propagate(g_hbm, src_hbm, dst_hbm, zeros_hbm, out_hbm,
                  src_v, dst_v, rows0_v, rows1_v, stage_v, acc_sh,
                  sem0, sem1):
    c = lax.axis_index("c")
    s = lax.axis_index("s")
    wid = s * NC + c

    pltpu.sync_copy(src_hbm.at[wid], src_v)
    pltpu.sync_copy(dst_hbm.at[wid], dst_v)
    pltpu.sync_copy(zeros_hbm, stage_v)
    pltpu.sync_copy(stage_v, acc_sh.at[pl.ds(s * RPS, RPS)])
    plsc.subcore_barrier()

    pltpu.async_copy(g_hbm.at[src_v.at[0]], rows0_v, sem0)

    def body(i, carry):
        j = 2 * i
        pltpu.async_copy(g_hbm.at[src_v.at[j + 1]], rows1_v, sem1)
        pltpu.make_async_copy(g_hbm.at[src_v.at[j]], rows0_v, sem0).wait()
        pltpu.sync_copy(rows0_v, acc_sh.at[dst_v.at[j]], add=True)
        pltpu.async_copy(g_hbm.at[src_v.at[j + 2]], rows0_v, sem0)
        pltpu.make_async_copy(g_hbm.at[src_v.at[j + 1]], rows1_v, sem1).wait()
        pltpu.sync_copy(rows1_v, acc_sh.at[dst_v.at[j + 1]], add=True)
        return carry

    lax.fori_loop(0, (K - 2) // 2, body, 0)
    pltpu.async_copy(g_hbm.at[src_v.at[K - 1]], rows1_v, sem1)
    pltpu.make_async_copy(g_hbm.at[src_v.at[K - 2]], rows0_v, sem0).wait()
    pltpu.sync_copy(rows0_v, acc_sh.at[dst_v.at[K - 2]], add=True)
    pltpu.make_async_copy(g_hbm.at[src_v.at[K - 1]], rows1_v, sem1).wait()
    pltpu.sync_copy(rows1_v, acc_sh.at[dst_v.at[K - 1]], add=True)
    plsc.subcore_barrier()

    pltpu.sync_copy(acc_sh.at[pl.ds(s * RPS, RPS)], stage_v)
    pltpu.sync_copy(stage_v, out_hbm.at[c, pl.ds(s * RPS, RPS)])


def _tc_first(x_ref, w_ref, degp_ref, g_ref, dinv_ref):
    deg = degp_ref[0] + degp_ref[1] + 1.0
    dinv = lax.rsqrt(deg)
    dinv_ref[...] = dinv
    h = jnp.dot(x_ref[...], w_ref[...], preferred_element_type=jnp.float32)
    g_ref[...] = h * dinv


def _tc_mid(p_ref, g0_ref, dinv_ref, b1_ref, wc_ref, g1_ref):
    dinv = dinv_ref[...]
    h1 = dinv * (p_ref[0] + p_ref[1] + g0_ref[...]) + b1_ref[...]
    h1 = jnp.maximum(h1, 0.0)
    mc = jnp.dot(h1, wc_ref[...], preferred_element_type=jnp.float32)
    g1_ref[...] = mc * dinv


def _tc_last(q_ref, g1_ref, dinv_ref, bc_ref, out_ref):
    out_ref[...] = (dinv_ref[...] * (q_ref[0] + q_ref[1] + g1_ref[...])
                    + bc_ref[...])


def kernel(x, edge_index, W1, b1, W2, b2, W3, b3):
    f32 = jnp.float32
    src = edge_index[0].astype(jnp.int32)
    dst = edge_index[1].astype(jnp.int32)
    e = src.shape[0]
    npad = E_PAD - e
    pad_dst = N + (jnp.arange(npad, dtype=jnp.int32) % (N_PAD - N))
    src_r = jnp.concatenate(
        [src, jnp.zeros((npad,), jnp.int32)]).reshape(NW, K, B)
    dst_r = jnp.concatenate([dst, pad_dst]).reshape(NW, K, B)

    x_p = jnp.pad(x, ((0, N_PAD - N), (0, 0)))
    ones_b = jnp.ones((B,), f32)
    zeros_1 = jnp.zeros((RPS,), f32)
    zeros_h = jnp.zeros((RPS, HID), f32)
    wc = jnp.concatenate([W2, W3], axis=1)
    bc = jnp.concatenate([b2, b3]).reshape(1, HID)
    b1r = b1.reshape(1, HID)

    degp = _sc_degree(dst_r, ones_b, zeros_1)

    g0, dinv = pl.pallas_call(
        _tc_first,
        out_shape=[
            jax.ShapeDtypeStruct((N_PAD, HID), f32),
            jax.ShapeDtypeStruct((N_PAD, 1), f32),
        ],
    )(x_p, W1, degp.reshape(NC, N_PAD, 1))

    p = _sc_propagate(g0, src_r, dst_r, zeros_h)

    g1 = pl.pallas_call(
        _tc_mid,
        out_shape=jax.ShapeDtypeStruct((N_PAD, HID), f32),
    )(p, g0, dinv, b1r, wc)

    q = _sc_propagate(g1, src_r, dst_r, zeros_h)

    outc = pl.pallas_call(
        _tc_last,
        out_shape=jax.ShapeDtypeStruct((N_PAD, HID), f32),
    )(q, g1, dinv, bc)

    mu = outc[:N, : HID // 2]
    sigma = outc[:N, HID // 2:]
    return (mu, sigma)


# speedup vs baseline: 1.7856x; 1.7856x over previous
import functools

import jax
import jax.numpy as jnp
from jax import lax
from jax.experimental import pallas as pl
from jax.experimental.pallas import tpu as pltpu
from jax.experimental.pallas import tpu_sc as plsc

N = 10000
F_IN = 128
HID = 32

NC, NS = 2, 16
NW = NC * NS
B = 128
K = 80
EPW = K * B
E_PAD = NW * EPW
N_PAD = 10240
RPS = N_PAD // NS

_mesh = plsc.VectorSubcoreMesh(
    core_axis_name="c", subcore_axis_name="s", num_cores=NC, num_subcores=NS)


@functools.partial(
    pl.kernel,
    out_type=jax.ShapeDtypeStruct((NC, N_PAD), jnp.float32),
    mesh=_mesh,
    scratch_types=[
        pltpu.VMEM((K, B), jnp.int32),
        pltpu.VMEM((B,), jnp.float32),
        pltpu.VMEM((RPS,), jnp.float32),
        pltpu.VMEM_SHARED((N_PAD,), jnp.float32),
        pltpu.SemaphoreType.DMA,
    ],
    compiler_params=pltpu.CompilerParams(use_tc_tiling_on_sc=False),
)
def _sc_degree(dst_hbm, ones_hbm, zeros_hbm, out_hbm, dst_v, ones_v, stage_v,
               acc_sh, sem):
    c = lax.axis_index("c")
    s = lax.axis_index("s")
    wid = s * NC + c

    pltpu.sync_copy(dst_hbm.at[wid], dst_v)
    pltpu.sync_copy(ones_hbm, ones_v)
    pltpu.sync_copy(zeros_hbm, stage_v)
    pltpu.sync_copy(stage_v, acc_sh.at[pl.ds(s * RPS, RPS)])
    plsc.subcore_barrier()

    def fire(j, carry):
        pltpu.async_copy(ones_v, acc_sh.at[dst_v.at[j]], sem, add=True)
        return carry

    def drain(j, carry):
        pltpu.make_async_copy(ones_v, acc_sh.at[dst_v.at[j]], sem).wait()
        return carry

    lax.fori_loop(0, K, fire, 0)
    lax.fori_loop(0, K, drain, 0)
    plsc.subcore_barrier()

    pltpu.sync_copy(acc_sh.at[pl.ds(s * RPS, RPS)], stage_v)
    pltpu.sync_copy(stage_v, out_hbm.at[c, pl.ds(s * RPS, RPS)])


@functools.partial(
    pl.kernel,
    out_type=jax.ShapeDtypeStruct((NC, N_PAD, HID), jnp.float32),
    mesh=_mesh,
    scratch_types=[
        pltpu.VMEM((K, B), jnp.int32),
        pltpu.VMEM((K, B), jnp.int32),
        pltpu.VMEM((B, HID), jnp.float32),
        pltpu.VMEM((B, HID), jnp.float32),
        pltpu.VMEM((RPS, HID), jnp.float32),
        pltpu.VMEM_SHARED((N_PAD, HID), jnp.float32),
        pltpu.VMEM_SHARED((N_PAD, HID), jnp.float32),
        pltpu.SemaphoreType.DMA,
        pltpu.SemaphoreType.DMA,
    ],
    compiler_params=pltpu.CompilerParams(use_tc_tiling_on_sc=False),
)
def _sc_propagate(g_hbm, src_hbm, dst_hbm, zeros_hbm, out_hbm,
                  src_v, dst_v, rows0_v, rows1_v, stage_v, acc_sh, g_sh,
                  sem0, sem1):
    c = lax.axis_index("c")
    s = lax.axis_index("s")
    wid = s * NC + c

    pltpu.sync_copy(src_hbm.at[wid], src_v)
    pltpu.sync_copy(dst_hbm.at[wid], dst_v)
    pltpu.sync_copy(g_hbm.at[pl.ds(s * RPS, RPS)], stage_v)
    pltpu.sync_copy(stage_v, g_sh.at[pl.ds(s * RPS, RPS)])
    pltpu.sync_copy(zeros_hbm, stage_v)
    pltpu.sync_copy(stage_v, acc_sh.at[pl.ds(s * RPS, RPS)])
    plsc.subcore_barrier()

    pltpu.async_copy(g_sh.at[src_v.at[0]], rows0_v, sem0)

    def body(i, carry):
        j = 2 * i
        pltpu.async_copy(g_sh.at[src_v.at[j + 1]], rows1_v, sem1)
        pltpu.make_async_copy(g_sh.at[src_v.at[j]], rows0_v, sem0).wait()
        pltpu.sync_copy(rows0_v, acc_sh.at[dst_v.at[j]], add=True)
        pltpu.async_copy(g_sh.at[src_v.at[j + 2]], rows0_v, sem0)
        pltpu.make_async_copy(g_sh.at[src_v.at[j + 1]], rows1_v, sem1).wait()
        pltpu.sync_copy(rows1_v, acc_sh.at[dst_v.at[j + 1]], add=True)
        return carry

    lax.fori_loop(0, (K - 2) // 2, body, 0)
    pltpu.async_copy(g_sh.at[src_v.at[K - 1]], rows1_v, sem1)
    pltpu.make_async_copy(g_sh.at[src_v.at[K - 2]], rows0_v, sem0).wait()
    pltpu.sync_copy(rows0_v, acc_sh.at[dst_v.at[K - 2]], add=True)
    pltpu.make_async_copy(g_sh.at[src_v.at[K - 1]], rows1_v, sem1).wait()
    pltpu.sync_copy(rows1_v, acc_sh.at[dst_v.at[K - 1]], add=True)
    plsc.subcore_barrier()

    pltpu.sync_copy(acc_sh.at[pl.ds(s * RPS, RPS)], stage_v)
    pltpu.sync_copy(stage_v, out_hbm.at[c, pl.ds(s * RPS, RPS)])


NROW = N_PAD // 128


def _lanes_to_column(v2d):
    row_id = lax.broadcasted_iota(jnp.int32, (N_PAD, NROW), 0) // 128
    col_id = lax.broadcasted_iota(jnp.int32, (N_PAD, NROW), 1)
    expand = (row_id == col_id).astype(jnp.float32)
    rows = jnp.dot(expand, v2d, preferred_element_type=jnp.float32)
    lane = lax.broadcasted_iota(jnp.int32, (N_PAD, 128), 1)
    n_mod = lax.broadcasted_iota(jnp.int32, (N_PAD, 128), 0) % 128
    sel = jnp.where(lane == n_mod, rows, 0.0)
    return jnp.sum(sel, axis=1, keepdims=True)


def _tc_first(x_ref, w_ref, degp_ref, g_ref, dinv_ref):
    deg2d = degp_ref[0] + degp_ref[1] + 1.0
    dinv2d = lax.rsqrt(deg2d)
    dinv_ref[...] = dinv2d
    dinv = _lanes_to_column(dinv2d)
    h = jnp.dot(x_ref[...], w_ref[...], preferred_element_type=jnp.float32)
    g_ref[...] = h * dinv


def _tc_mid(p_ref, g0_ref, dinv_ref, b1_ref, wc_ref, g1_ref):
    dinv = _lanes_to_column(dinv_ref[...])
    h1 = dinv * (p_ref[0] + p_ref[1] + g0_ref[...]) + b1_ref[...]
    h1 = jnp.maximum(h1, 0.0)
    mc = jnp.dot(h1, wc_ref[...], preferred_element_type=jnp.float32)
    g1_ref[...] = mc * dinv


def _tc_last(q_ref, g1_ref, dinv_ref, bc_ref, out_ref):
    dinv = _lanes_to_column(dinv_ref[...])
    out_ref[...] = (dinv * (q_ref[0] + q_ref[1] + g1_ref[...])
                    + bc_ref[...])


def kernel(x, edge_index, W1, b1, W2, b2, W3, b3):
    f32 = jnp.float32
    src = edge_index[0].astype(jnp.int32)
    dst = edge_index[1].astype(jnp.int32)
    e = src.shape[0]
    npad = E_PAD - e
    pad_dst = N + (jnp.arange(npad, dtype=jnp.int32) % (N_PAD - N))
    src_r = jnp.concatenate(
        [src, jnp.zeros((npad,), jnp.int32)]).reshape(NW, K, B)
    dst_r = jnp.concatenate([dst, pad_dst]).reshape(NW, K, B)

    x_p = jnp.pad(x, ((0, N_PAD - N), (0, 0)))
    ones_b = jnp.ones((B,), f32)
    zeros_1 = jnp.zeros((RPS,), f32)
    zeros_h = jnp.zeros((RPS, HID), f32)
    wc = jnp.concatenate([W2, W3], axis=1)
    bc = jnp.concatenate([b2, b3]).reshape(1, HID)
    b1r = b1.reshape(1, HID)

    degp = _sc_degree(dst_r, ones_b, zeros_1)

    g0, dinv = pl.pallas_call(
        _tc_first,
        out_shape=[
            jax.ShapeDtypeStruct((N_PAD, HID), f32),
            jax.ShapeDtypeStruct((N_PAD // 128, 128), f32),
        ],
    )(x_p, W1, degp.reshape(NC, N_PAD // 128, 128))

    p = _sc_propagate(g0, src_r, dst_r, zeros_h)

    g1 = pl.pallas_call(
        _tc_mid,
        out_shape=jax.ShapeDtypeStruct((N_PAD, HID), f32),
    )(p, g0, dinv, b1r, wc)

    q = _sc_propagate(g1, src_r, dst_r, zeros_h)

    outc = pl.pallas_call(
        _tc_last,
        out_shape=jax.ShapeDtypeStruct((N_PAD, HID), f32),
    )(q, g1, dinv, bc)

    mu = outc[:N, : HID // 2]
    sigma = outc[:N, HID // 2:]
    return (mu, sigma)
